# baseline (speedup 1.0000x reference)
import functools

import jax
import jax.numpy as jnp
from jax import lax
from jax.experimental import pallas as pl
from jax.experimental.pallas import tpu as pltpu
from jax.experimental.pallas import tpu_sc as plsc

N = 10000
D = 128
NC = 2
NS = 16
NW = NC * NS
CHUNK = 128
NCH = 80
EPT = NCH * CHUNK
E_PAD = NW * EPT
ACC_N = 10240
RPT = ACC_N // NS

_mesh = plsc.VectorSubcoreMesh(core_axis_name="c", subcore_axis_name="s")


DEGW = 128


@functools.partial(
    pl.kernel,
    out_type=jax.ShapeDtypeStruct((NC, ACC_N, DEGW), jnp.float32),
    mesh=_mesh,
    scratch_types=[
        pltpu.VMEM((NCH, CHUNK), jnp.int32),
        pltpu.VMEM((CHUNK, DEGW), jnp.float32),
        pltpu.VMEM_SHARED((ACC_N, DEGW), jnp.float32),
    ],
)
def _sc_degree(dst_hbm, ones_hbm, zeros_hbm, out_hbm, dst_v, ones_v, acc_sh):
    c = lax.axis_index("c")
    s = lax.axis_index("s")
    w = c * NS + s
    pltpu.sync_copy(zeros_hbm, acc_sh.at[pl.ds(s * RPT, 320)])
    pltpu.sync_copy(zeros_hbm, acc_sh.at[pl.ds(s * RPT + 320, 320)])
    pltpu.sync_copy(ones_hbm, ones_v)
    pltpu.sync_copy(dst_hbm.at[pl.ds(w * NCH, NCH)], dst_v)
    plsc.subcore_barrier()

    def body(j, carry):
        pltpu.sync_copy(ones_v, acc_sh.at[dst_v.at[j]], add=True)
        return carry

    lax.fori_loop(0, NCH, body, 0)
    plsc.subcore_barrier()
    pltpu.sync_copy(acc_sh.at[pl.ds(s * RPT, RPT)],
                    out_hbm.at[c, pl.ds(s * RPT, RPT)])


NBUF = 2
SSZ = 32
SGRP = SSZ // NBUF
FAST_C = 0
NCH_F = 160
NCH_S = 0


@functools.partial(
    pl.kernel,
    out_type=jax.ShapeDtypeStruct((NC, ACC_N, D), jnp.float32),
    mesh=_mesh,
    scratch_types=[
        pltpu.VMEM((SSZ, CHUNK), jnp.int32),
        pltpu.VMEM((SSZ, CHUNK), jnp.int32),
        [pltpu.VMEM((CHUNK, D), jnp.float32)] * NBUF,
        pltpu.VMEM_SHARED((ACC_N, D), jnp.float32),
        [pltpu.SemaphoreType.DMA] * NBUF,
    ],
)
def _sc_aggregate(g_hbm, src_hbm, dst_hbm, zeros_hbm, out_hbm,
                  src_v, dst_v, rows_v, acc_sh, gsems):
    c = lax.axis_index("c")
    s = lax.axis_index("s")
    is_fast = c == FAST_C
    n_stages = jnp.where(is_fast, NCH_F // SSZ, NCH_S // SSZ)
    base_row = s * NCH_F
    pltpu.sync_copy(zeros_hbm, acc_sh.at[pl.ds(s * RPT, 320)])
    pltpu.sync_copy(zeros_hbm, acc_sh.at[pl.ds(s * RPT + 320, 320)])
    plsc.subcore_barrier()

    def stage(qi, carry):
        row0 = base_row + qi * SSZ
        pltpu.sync_copy(src_hbm.at[pl.ds(row0, SSZ)], src_v)
        pltpu.sync_copy(dst_hbm.at[pl.ds(row0, SSZ)], dst_v)

        for b in range(NBUF):
            pltpu.async_copy(g_hbm.at[src_v.at[b]], rows_v[b], gsems[b])

        def step(j, b):
            pltpu.make_async_copy(g_hbm.at[src_v.at[j]], rows_v[b],
                                  gsems[b]).wait()
            pltpu.sync_copy(rows_v[b], acc_sh.at[dst_v.at[j]], add=True)

        def group(gi, carry2):
            for b in range(NBUF):
                j = gi * NBUF + b
                step(j, b)
                pltpu.async_copy(g_hbm.at[src_v.at[j + NBUF]], rows_v[b],
                                 gsems[b])
            return carry2

        lax.fori_loop(0, SGRP - 1, group, 0)
        for b in range(NBUF):
            step(SSZ - NBUF + b, b)
        return carry

    lax.fori_loop(0, n_stages, stage, 0)

    plsc.subcore_barrier()
    pltpu.sync_copy(acc_sh.at[pl.ds(s * RPT, RPT)],
                    out_hbm.at[c, pl.ds(s * RPT, RPT)])


def _dinv_body(degp_ref, out_ref):
    deg = jnp.sum(degp_ref[...], axis=(0, 2)) * (1.0 / DEGW) + 1.0
    out_ref[...] = lax.rsqrt(deg)[None, :]


def _tc_dinv(deg_partials):
    return pl.pallas_call(
        _dinv_body,
        out_shape=jax.ShapeDtypeStruct((1, ACC_N), jnp.float32),
    )(deg_partials)


_R = 2000


def _lin_body(x_ref, w_ref, dinv_ref, h_ref, g_ref):
    h = jnp.dot(x_ref[...], w_ref[...], preferred_element_type=jnp.float32)
    h_ref[...] = h
    g_ref[...] = h * dinv_ref[...]


def _tc_layer_in(x, w, dinv_col):
    return pl.pallas_call(
        _lin_body,
        grid=(N // _R,),
        in_specs=[
            pl.BlockSpec((_R, D), lambda i: (i, 0)),
            pl.BlockSpec((D, D), lambda i: (0, 0)),
            pl.BlockSpec((_R, 1), lambda i: (i, 0)),
        ],
        out_specs=[
            pl.BlockSpec((_R, D), lambda i: (i, 0)),
            pl.BlockSpec((_R, D), lambda i: (i, 0)),
        ],
        out_shape=[
            jax.ShapeDtypeStruct((N, D), jnp.float32),
            jax.ShapeDtypeStruct((N, D), jnp.float32),
        ],
    )(x, w, dinv_col)


def _mid_body(p_ref, h1_ref, dinv_ref, w_ref, h2_ref, g2_ref):
    dv = dinv_ref[...]
    agg = p_ref[0] + p_ref[1]
    z = jnp.maximum(dv * agg + dv * dv * h1_ref[...], 0.0)
    h2 = jnp.dot(z, w_ref[...], preferred_element_type=jnp.float32)
    h2_ref[...] = h2
    g2_ref[...] = dv * h2


def _tc_mid(p, h1, dinv_col, w2):
    return pl.pallas_call(
        _mid_body,
        grid=(N // _R,),
        in_specs=[
            pl.BlockSpec((NC, _R, D), lambda i: (0, i, 0)),
            pl.BlockSpec((_R, D), lambda i: (i, 0)),
            pl.BlockSpec((_R, 1), lambda i: (i, 0)),
            pl.BlockSpec((D, D), lambda i: (0, 0)),
        ],
        out_specs=[
            pl.BlockSpec((_R, D), lambda i: (i, 0)),
            pl.BlockSpec((_R, D), lambda i: (i, 0)),
        ],
        out_shape=[
            jax.ShapeDtypeStruct((N, D), jnp.float32),
            jax.ShapeDtypeStruct((N, D), jnp.float32),
        ],
    )(p, h1, dinv_col, w2)


def _out_body(p_ref, h2_ref, dinv_ref, b_ref, o_ref):
    dv = dinv_ref[...]
    o_ref[...] = dv * (p_ref[0] + p_ref[1]) + dv * dv * h2_ref[...] + b_ref[...]


def _tc_out(p, h2, dinv_col, b):
    return pl.pallas_call(
        _out_body,
        grid=(N // _R,),
        in_specs=[
            pl.BlockSpec((NC, _R, D), lambda i: (0, i, 0)),
            pl.BlockSpec((_R, D), lambda i: (i, 0)),
            pl.BlockSpec((_R, 1), lambda i: (i, 0)),
            pl.BlockSpec((1, D), lambda i: (0, 0)),
        ],
        out_specs=pl.BlockSpec((_R, D), lambda i: (i, 0)),
        out_shape=jax.ShapeDtypeStruct((N, D), jnp.float32),
    )(p, h2, dinv_col, b)


def kernel(x, edge_index, W1, W2, b2):
    E = edge_index.shape[1]
    pad = E_PAD - E
    src = jnp.concatenate(
        [edge_index[0].astype(jnp.int32), jnp.zeros((pad,), jnp.int32)])
    dst = jnp.concatenate(
        [edge_index[1].astype(jnp.int32),
         N + (jnp.arange(pad, dtype=jnp.int32) % (ACC_N - N))])
    src2d = src.reshape(NW * NCH, CHUNK)
    dst2d = dst.reshape(NW * NCH, CHUNK)
    zeros = jnp.zeros((320, D), jnp.float32)
    ones_deg = jnp.ones((CHUNK, DEGW), jnp.float32)

    deg_p = _sc_degree(dst2d, ones_deg, zeros)
    dinv2d = _tc_dinv(deg_p)
    dinv_col = dinv2d.reshape(ACC_N)[:N, None]

    h1, g1 = _tc_layer_in(x, W1, dinv_col)
    p1 = _sc_aggregate(g1, src2d, dst2d, zeros)
    h2, g2 = _tc_mid(p1, h1, dinv_col, W2)
    p2 = _sc_aggregate(g2, src2d, dst2d, zeros)
    out = _tc_out(p2, h2, dinv_col, b2.reshape(1, D))
    return out


# speedup vs baseline: 3.5674x; 3.5674x over previous
import functools

import jax
import jax.numpy as jnp
from jax import lax
from jax.experimental import pallas as pl
from jax.experimental.pallas import tpu as pltpu
from jax.experimental.pallas import tpu_sc as plsc

N = 10000
D = 128
NC = 2
NS = 16
NW = NC * NS
CHUNK = 112
NCH = 96
E_PAD = NW * NCH * CHUNK
ACC_N = 10240
RPT = ACC_N // NS

_mesh = plsc.VectorSubcoreMesh(core_axis_name="c", subcore_axis_name="s")


DEGW = 128


@functools.partial(
    pl.kernel,
    out_type=jax.ShapeDtypeStruct((NC, ACC_N, DEGW), jnp.float32),
    mesh=_mesh,
    scratch_types=[
        pltpu.VMEM((NCH, CHUNK), jnp.int32),
        pltpu.VMEM((CHUNK, DEGW), jnp.float32),
        pltpu.VMEM_SHARED((ACC_N, DEGW), jnp.float32),
    ],
)
def _sc_degree(dst_hbm, ones_hbm, zeros_hbm, out_hbm, dst_v, ones_v, acc_sh):
    c = lax.axis_index("c")
    s = lax.axis_index("s")
    w = c * NS + s
    pltpu.sync_copy(zeros_hbm, acc_sh.at[pl.ds(s * RPT, 320)])
    pltpu.sync_copy(zeros_hbm, acc_sh.at[pl.ds(s * RPT + 320, 320)])
    pltpu.sync_copy(ones_hbm, ones_v)
    pltpu.sync_copy(dst_hbm.at[pl.ds(w * NCH, NCH)], dst_v)
    plsc.subcore_barrier()

    def body(j, carry):
        pltpu.sync_copy(ones_v, acc_sh.at[dst_v.at[j]], add=True)
        return carry

    lax.fori_loop(0, NCH, body, 0)
    plsc.subcore_barrier()
    pltpu.sync_copy(acc_sh.at[pl.ds(s * RPT, RPT)],
                    out_hbm.at[c, pl.ds(s * RPT, RPT)])


NBUF = 3
SSZ = 24
NSTG = NCH // SSZ


@functools.partial(
    pl.kernel,
    out_type=jax.ShapeDtypeStruct((NC, ACC_N, D), jnp.float32),
    mesh=_mesh,
    scratch_types=[
        pltpu.VMEM((SSZ, CHUNK), jnp.int32),
        pltpu.VMEM((SSZ, CHUNK), jnp.int32),
        [pltpu.VMEM((CHUNK, D), jnp.float32)] * NBUF,
        pltpu.VMEM_SHARED((ACC_N, D), jnp.float32),
        [pltpu.SemaphoreType.DMA] * NBUF,
        [pltpu.SemaphoreType.DMA] * NBUF,
    ],
)
def _sc_aggregate(g_hbm, src_hbm, dst_hbm, zeros_hbm, out_hbm,
                  src_v, dst_v, rows_v, acc_sh, gsems, ssems):
    c = lax.axis_index("c")
    s = lax.axis_index("s")
    base_row = (c * NS + s) * NCH
    pltpu.sync_copy(zeros_hbm, acc_sh.at[pl.ds(s * RPT, 320)])
    pltpu.sync_copy(zeros_hbm, acc_sh.at[pl.ds(s * RPT + 320, 320)])
    plsc.subcore_barrier()

    def stage(qi, carry):
        row0 = base_row + qi * SSZ
        pltpu.sync_copy(src_hbm.at[pl.ds(row0, SSZ)], src_v)
        pltpu.sync_copy(dst_hbm.at[pl.ds(row0, SSZ)], dst_v)

        for b in range(NBUF):
            pltpu.async_copy(g_hbm.at[src_v.at[b]], rows_v[b], gsems[b])
        for j in range(SSZ):
            b = j % NBUF
            pltpu.make_async_copy(g_hbm.at[src_v.at[j]], rows_v[b],
                                  gsems[b]).wait()
            pltpu.async_copy(rows_v[b], acc_sh.at[dst_v.at[j]], ssems[b],
                             add=True)
            if j >= 1 and j + 2 < SSZ:
                bn = (j - 1) % NBUF
                pltpu.make_async_copy(rows_v[bn], acc_sh.at[dst_v.at[j - 1]],
                                      ssems[bn]).wait()
                pltpu.async_copy(g_hbm.at[src_v.at[j + 2]], rows_v[bn],
                                 gsems[bn])
        for j in range(SSZ - NBUF, SSZ):
            b = j % NBUF
            pltpu.make_async_copy(rows_v[b], acc_sh.at[dst_v.at[j]],
                                  ssems[b]).wait()
        return carry

    lax.fori_loop(0, NSTG, stage, 0)

    plsc.subcore_barrier()
    pltpu.sync_copy(acc_sh.at[pl.ds(s * RPT, RPT)],
                    out_hbm.at[c, pl.ds(s * RPT, RPT)])


def _dinv_body(degp_ref, out_ref):
    deg = jnp.sum(degp_ref[...], axis=(0, 2)) * (1.0 / DEGW) + 1.0
    out_ref[...] = lax.rsqrt(deg)[None, :]


def _tc_dinv(deg_partials):
    return pl.pallas_call(
        _dinv_body,
        out_shape=jax.ShapeDtypeStruct((1, ACC_N), jnp.float32),
    )(deg_partials)


_R = 2000


def _lin_body(x_ref, w_ref, dinv_ref, h_ref, g_ref):
    h = jnp.dot(x_ref[...], w_ref[...], preferred_element_type=jnp.float32)
    h_ref[...] = h
    g_ref[...] = h * dinv_ref[...]


def _tc_layer_in(x, w, dinv_col):
    return pl.pallas_call(
        _lin_body,
        grid=(N // _R,),
        in_specs=[
            pl.BlockSpec((_R, D), lambda i: (i, 0)),
            pl.BlockSpec((D, D), lambda i: (0, 0)),
            pl.BlockSpec((_R, 1), lambda i: (i, 0)),
        ],
        out_specs=[
            pl.BlockSpec((_R, D), lambda i: (i, 0)),
            pl.BlockSpec((_R, D), lambda i: (i, 0)),
        ],
        out_shape=[
            jax.ShapeDtypeStruct((N, D), jnp.float32),
            jax.ShapeDtypeStruct((N, D), jnp.float32),
        ],
    )(x, w, dinv_col)


def _mid_body(p_ref, h1_ref, dinv_ref, w_ref, h2_ref, g2_ref):
    dv = dinv_ref[...]
    agg = p_ref[0] + p_ref[1]
    z = jnp.maximum(dv * agg + dv * dv * h1_ref[...], 0.0)
    h2 = jnp.dot(z, w_ref[...], preferred_element_type=jnp.float32)
    h2_ref[...] = h2
    g2_ref[...] = dv * h2


def _tc_mid(p, h1, dinv_col, w2):
    return pl.pallas_call(
        _mid_body,
        grid=(N // _R,),
        in_specs=[
            pl.BlockSpec((NC, _R, D), lambda i: (0, i, 0)),
            pl.BlockSpec((_R, D), lambda i: (i, 0)),
            pl.BlockSpec((_R, 1), lambda i: (i, 0)),
            pl.BlockSpec((D, D), lambda i: (0, 0)),
        ],
        out_specs=[
            pl.BlockSpec((_R, D), lambda i: (i, 0)),
            pl.BlockSpec((_R, D), lambda i: (i, 0)),
        ],
        out_shape=[
            jax.ShapeDtypeStruct((N, D), jnp.float32),
            jax.ShapeDtypeStruct((N, D), jnp.float32),
        ],
    )(p, h1, dinv_col, w2)


def _out_body(p_ref, h2_ref, dinv_ref, b_ref, o_ref):
    dv = dinv_ref[...]
    o_ref[...] = dv * (p_ref[0] + p_ref[1]) + dv * dv * h2_ref[...] + b_ref[...]


def _tc_out(p, h2, dinv_col, b):
    return pl.pallas_call(
        _out_body,
        grid=(N // _R,),
        in_specs=[
            pl.BlockSpec((NC, _R, D), lambda i: (0, i, 0)),
            pl.BlockSpec((_R, D), lambda i: (i, 0)),
            pl.BlockSpec((_R, 1), lambda i: (i, 0)),
            pl.BlockSpec((1, D), lambda i: (0, 0)),
        ],
        out_specs=pl.BlockSpec((_R, D), lambda i: (i, 0)),
        out_shape=jax.ShapeDtypeStruct((N, D), jnp.float32),
    )(p, h2, dinv_col, b)


def kernel(x, edge_index, W1, W2, b2):
    E = edge_index.shape[1]
    pad = E_PAD - E
    src = jnp.concatenate(
        [edge_index[0].astype(jnp.int32),
         jnp.arange(pad, dtype=jnp.int32) % N])
    dst = jnp.concatenate(
        [edge_index[1].astype(jnp.int32),
         N + (jnp.arange(pad, dtype=jnp.int32) % (ACC_N - N))])
    src2d = src.reshape(NW * NCH, CHUNK)
    dst2d = dst.reshape(NW * NCH, CHUNK)
    zeros = jnp.zeros((320, D), jnp.float32)
    ones_deg = jnp.ones((CHUNK, DEGW), jnp.float32)

    deg_p = _sc_degree(dst2d, ones_deg, zeros)
    dinv2d = _tc_dinv(deg_p)
    dinv_col = dinv2d.reshape(ACC_N)[:N, None]

    h1, g1 = _tc_layer_in(x, W1, dinv_col)
    p1 = _sc_aggregate(g1, src2d, dst2d, zeros)
    h2, g2 = _tc_mid(p1, h1, dinv_col, W2)
    p2 = _sc_aggregate(g2, src2d, dst2d, zeros)
    out = _tc_out(p2, h2, dinv_col, b2.reshape(1, D))
    return out


# speedup vs baseline: 3.8228x; 1.0716x over previous
import functools

import jax
import jax.numpy as jnp
from jax import lax
from jax.experimental import pallas as pl
from jax.experimental.pallas import tpu as pltpu
from jax.experimental.pallas import tpu_sc as plsc

N = 10000
D = 128
NC = 2
NS = 16
NW = NC * NS
CHUNK = 128
NCH = 80
EPT = NCH * CHUNK
E_PAD = NW * EPT
ACC_N = 10240
RPT = ACC_N // NS

_mesh = plsc.VectorSubcoreMesh(core_axis_name="c", subcore_axis_name="s")


KB = 4
DEGW = 128


@functools.partial(
    pl.kernel,
    out_type=jax.ShapeDtypeStruct((NC, ACC_N, DEGW), jnp.float32),
    mesh=_mesh,
    scratch_types=[
        pltpu.VMEM((NCH, CHUNK), jnp.int32),
        pltpu.VMEM((CHUNK, DEGW), jnp.float32),
        pltpu.VMEM_SHARED((ACC_N, DEGW), jnp.float32),
        [pltpu.SemaphoreType.DMA] * KB,
    ],
)
def _sc_degree(dst_hbm, ones_hbm, zeros_hbm, out_hbm, dst_v, ones_v, acc_sh,
               dsems):
    c = lax.axis_index("c")
    s = lax.axis_index("s")
    w = c * NS + s
    pltpu.sync_copy(zeros_hbm, acc_sh.at[pl.ds(s * RPT, 320)])
    pltpu.sync_copy(zeros_hbm, acc_sh.at[pl.ds(s * RPT + 320, 320)])
    pltpu.sync_copy(ones_hbm, ones_v)
    pltpu.sync_copy(dst_hbm.at[pl.ds(w * NCH, NCH)], dst_v)
    plsc.subcore_barrier()

    for b in range(KB):
        pltpu.async_copy(ones_v, acc_sh.at[dst_v.at[b]], dsems[b], add=True)

    def group(gi, carry):
        for b in range(KB):
            j = (gi + 1) * KB + b
            pltpu.make_async_copy(ones_v, acc_sh.at[dst_v.at[j - KB]],
                                  dsems[b]).wait()
            pltpu.async_copy(ones_v, acc_sh.at[dst_v.at[j]], dsems[b],
                             add=True)
        return carry

    lax.fori_loop(0, NCH // KB - 1, group, 0)
    for b in range(KB):
        pltpu.make_async_copy(ones_v, acc_sh.at[dst_v.at[NCH - KB + b]],
                              dsems[b]).wait()
    plsc.subcore_barrier()
    pltpu.sync_copy(acc_sh.at[pl.ds(s * RPT, RPT)],
                    out_hbm.at[c, pl.ds(s * RPT, RPT)])


NBUF = 2
SSZ = 40
SGRP = SSZ // NBUF
NSTG = NCH // SSZ


@functools.partial(
    pl.kernel,
    out_type=jax.ShapeDtypeStruct((NC, ACC_N, D), jnp.float32),
    mesh=_mesh,
    scratch_types=[
        pltpu.VMEM((SSZ, CHUNK), jnp.int32),
        pltpu.VMEM((SSZ, CHUNK), jnp.int32),
        [pltpu.VMEM((CHUNK, D), jnp.float32)] * NBUF,
        pltpu.VMEM_SHARED((ACC_N, D), jnp.float32),
        [pltpu.SemaphoreType.DMA] * NBUF,
    ],
)
def _sc_aggregate(g_hbm, src_hbm, dst_hbm, zeros_hbm, out_hbm,
                  src_v, dst_v, rows_v, acc_sh, gsems):
    c = lax.axis_index("c")
    s = lax.axis_index("s")
    base_row = (c * NS + s) * NCH
    pltpu.sync_copy(zeros_hbm, acc_sh.at[pl.ds(s * RPT, 320)])
    pltpu.sync_copy(zeros_hbm, acc_sh.at[pl.ds(s * RPT + 320, 320)])
    plsc.subcore_barrier()

    def stage(qi, carry):
        row0 = base_row + qi * SSZ
        pltpu.sync_copy(src_hbm.at[pl.ds(row0, SSZ)], src_v)
        pltpu.sync_copy(dst_hbm.at[pl.ds(row0, SSZ)], dst_v)

        for b in range(NBUF):
            pltpu.async_copy(g_hbm.at[src_v.at[b]], rows_v[b], gsems[b])

        def step(j, b):
            pltpu.make_async_copy(g_hbm.at[src_v.at[j]], rows_v[b],
                                  gsems[b]).wait()
            pltpu.sync_copy(rows_v[b], acc_sh.at[dst_v.at[j]], add=True)

        def group(gi, carry2):
            for b in range(NBUF):
                j = gi * NBUF + b
                step(j, b)
                pltpu.async_copy(g_hbm.at[src_v.at[j + NBUF]], rows_v[b],
                                 gsems[b])
            return carry2

        lax.fori_loop(0, SGRP - 1, group, 0)
        for b in range(NBUF):
            step(SSZ - NBUF + b, b)
        return carry

    lax.fori_loop(0, NSTG, stage, 0)

    plsc.subcore_barrier()
    pltpu.sync_copy(acc_sh.at[pl.ds(s * RPT, RPT)],
                    out_hbm.at[c, pl.ds(s * RPT, RPT)])


def _dinv_body(degp_ref, out_ref):
    deg = jnp.sum(degp_ref[...], axis=(0, 2)) * (1.0 / DEGW) + 1.0
    out_ref[...] = lax.rsqrt(deg)[None, :]


def _tc_dinv(deg_partials):
    return pl.pallas_call(
        _dinv_body,
        out_shape=jax.ShapeDtypeStruct((1, ACC_N), jnp.float32),
    )(deg_partials)


_R = 2000


def _lin_body(x_ref, w_ref, dinv_ref, h_ref, g_ref):
    h = jnp.dot(x_ref[...], w_ref[...], preferred_element_type=jnp.float32)
    h_ref[...] = h
    g_ref[...] = h * dinv_ref[...]


def _tc_layer_in(x, w, dinv_col):
    return pl.pallas_call(
        _lin_body,
        grid=(N // _R,),
        in_specs=[
            pl.BlockSpec((_R, D), lambda i: (i, 0)),
            pl.BlockSpec((D, D), lambda i: (0, 0)),
            pl.BlockSpec((_R, 1), lambda i: (i, 0)),
        ],
        out_specs=[
            pl.BlockSpec((_R, D), lambda i: (i, 0)),
            pl.BlockSpec((_R, D), lambda i: (i, 0)),
        ],
        out_shape=[
            jax.ShapeDtypeStruct((N, D), jnp.float32),
            jax.ShapeDtypeStruct((N, D), jnp.float32),
        ],
    )(x, w, dinv_col)


def _mid_body(p_ref, h1_ref, dinv_ref, w_ref, h2_ref, g2_ref):
    dv = dinv_ref[...]
    agg = p_ref[0] + p_ref[1]
    z = jnp.maximum(dv * agg + dv * dv * h1_ref[...], 0.0)
    h2 = jnp.dot(z, w_ref[...], preferred_element_type=jnp.float32)
    h2_ref[...] = h2
    g2_ref[...] = dv * h2


def _tc_mid(p, h1, dinv_col, w2):
    return pl.pallas_call(
        _mid_body,
        grid=(N // _R,),
        in_specs=[
            pl.BlockSpec((NC, _R, D), lambda i: (0, i, 0)),
            pl.BlockSpec((_R, D), lambda i: (i, 0)),
            pl.BlockSpec((_R, 1), lambda i: (i, 0)),
            pl.BlockSpec((D, D), lambda i: (0, 0)),
        ],
        out_specs=[
            pl.BlockSpec((_R, D), lambda i: (i, 0)),
            pl.BlockSpec((_R, D), lambda i: (i, 0)),
        ],
        out_shape=[
            jax.ShapeDtypeStruct((N, D), jnp.float32),
            jax.ShapeDtypeStruct((N, D), jnp.float32),
        ],
    )(p, h1, dinv_col, w2)


def _out_body(p_ref, h2_ref, dinv_ref, b_ref, o_ref):
    dv = dinv_ref[...]
    o_ref[...] = dv * (p_ref[0] + p_ref[1]) + dv * dv * h2_ref[...] + b_ref[...]


def _tc_out(p, h2, dinv_col, b):
    return pl.pallas_call(
        _out_body,
        grid=(N // _R,),
        in_specs=[
            pl.BlockSpec((NC, _R, D), lambda i: (0, i, 0)),
            pl.BlockSpec((_R, D), lambda i: (i, 0)),
            pl.BlockSpec((_R, 1), lambda i: (i, 0)),
            pl.BlockSpec((1, D), lambda i: (0, 0)),
        ],
        out_specs=pl.BlockSpec((_R, D), lambda i: (i, 0)),
        out_shape=jax.ShapeDtypeStruct((N, D), jnp.float32),
    )(p, h2, dinv_col, b)


def kernel(x, edge_index, W1, W2, b2):
    E = edge_index.shape[1]
    pad = E_PAD - E
    src = jnp.concatenate(
        [edge_index[0].astype(jnp.int32),
         jnp.arange(pad, dtype=jnp.int32) % N])
    dst = jnp.concatenate(
        [edge_index[1].astype(jnp.int32),
         N + (jnp.arange(pad, dtype=jnp.int32) % (ACC_N - N))])
    src2d = src.reshape(NW * NCH, CHUNK)
    dst2d = dst.reshape(NW * NCH, CHUNK)
    zeros = jnp.zeros((320, D), jnp.float32)
    ones_deg = jnp.ones((CHUNK, DEGW), jnp.float32)

    deg_p = _sc_degree(dst2d, ones_deg, zeros)
    dinv2d = _tc_dinv(deg_p)
    dinv_col = dinv2d.reshape(ACC_N)[:N, None]

    h1, g1 = _tc_layer_in(x, W1, dinv_col)
    p1 = _sc_aggregate(g1, src2d, dst2d, zeros)
    h2, g2 = _tc_mid(p1, h1, dinv_col, W2)
    p2 = _sc_aggregate(g2, src2d, dst2d, zeros)
    out = _tc_out(p2, h2, dinv_col, b2.reshape(1, D))
    return out


# speedup vs baseline: 3.9535x; 1.0342x over previous
import functools

import jax
import jax.numpy as jnp
from jax import lax
from jax.experimental import pallas as pl
from jax.experimental.pallas import tpu as pltpu
from jax.experimental.pallas import tpu_sc as plsc

N = 10000
D = 128
NC = 2
NS = 16
NW = NC * NS
CHUNK = 128
NCH = 80
EPT = NCH * CHUNK
E_PAD = NW * EPT
ACC_N = 10240
RPT = ACC_N // NS

_mesh = plsc.VectorSubcoreMesh(core_axis_name="c", subcore_axis_name="s")


KB = 4
DEGW = 128


@functools.partial(
    pl.kernel,
    out_type=jax.ShapeDtypeStruct((NC, ACC_N, DEGW), jnp.float32),
    mesh=_mesh,
    scratch_types=[
        pltpu.VMEM((NCH, CHUNK), jnp.int32),
        pltpu.VMEM((CHUNK, DEGW), jnp.float32),
        pltpu.VMEM_SHARED((ACC_N, DEGW), jnp.float32),
        [pltpu.SemaphoreType.DMA] * KB,
    ],
)
def _sc_degree(dst_hbm, ones_hbm, zeros_hbm, out_hbm, dst_v, ones_v, acc_sh,
               dsems):
    c = lax.axis_index("c")
    s = lax.axis_index("s")
    w = c * NS + s
    pltpu.sync_copy(zeros_hbm, acc_sh.at[pl.ds(s * RPT, 320)])
    pltpu.sync_copy(zeros_hbm, acc_sh.at[pl.ds(s * RPT + 320, 320)])
    pltpu.sync_copy(ones_hbm, ones_v)
    pltpu.sync_copy(dst_hbm.at[pl.ds(w * NCH, NCH)], dst_v)
    plsc.subcore_barrier()

    for b in range(KB):
        pltpu.async_copy(ones_v, acc_sh.at[dst_v.at[b]], dsems[b], add=True)

    def group(gi, carry):
        for b in range(KB):
            j = (gi + 1) * KB + b
            pltpu.make_async_copy(ones_v, acc_sh.at[dst_v.at[j - KB]],
                                  dsems[b]).wait()
            pltpu.async_copy(ones_v, acc_sh.at[dst_v.at[j]], dsems[b],
                             add=True)
        return carry

    lax.fori_loop(0, NCH // KB - 1, group, 0)
    for b in range(KB):
        pltpu.make_async_copy(ones_v, acc_sh.at[dst_v.at[NCH - KB + b]],
                              dsems[b]).wait()
    plsc.subcore_barrier()
    pltpu.sync_copy(acc_sh.at[pl.ds(s * RPT, RPT)],
                    out_hbm.at[c, pl.ds(s * RPT, RPT)])


NBUF = 2
SSZ = 40
SGRP = SSZ // NBUF
NSTG = NCH // SSZ


@functools.partial(
    pl.kernel,
    out_type=jax.ShapeDtypeStruct((NC, ACC_N, D), jnp.float32),
    mesh=_mesh,
    scratch_types=[
        pltpu.VMEM((SSZ, CHUNK), jnp.int32),
        pltpu.VMEM((SSZ, CHUNK), jnp.int32),
        [pltpu.VMEM((CHUNK, D), jnp.float32)] * NBUF,
        pltpu.VMEM_SHARED((ACC_N, D), jnp.float32),
        [pltpu.SemaphoreType.DMA] * NBUF,
    ],
)
def _sc_aggregate(g_hbm, src_hbm, dst_hbm, zeros_hbm, out_hbm,
                  src_v, dst_v, rows_v, acc_sh, gsems):
    c = lax.axis_index("c")
    s = lax.axis_index("s")
    base_row = (c * NS + s) * NCH
    pltpu.sync_copy(zeros_hbm, acc_sh.at[pl.ds(s * RPT, 320)])
    pltpu.sync_copy(zeros_hbm, acc_sh.at[pl.ds(s * RPT + 320, 320)])
    plsc.subcore_barrier()

    def stage(qi, carry):
        row0 = base_row + qi * SSZ
        pltpu.sync_copy(src_hbm.at[pl.ds(row0, SSZ)], src_v)
        pltpu.sync_copy(dst_hbm.at[pl.ds(row0, SSZ)], dst_v)

        for b in range(NBUF):
            pltpu.async_copy(g_hbm.at[src_v.at[b]], rows_v[b], gsems[b])

        def step(j, b):
            pltpu.make_async_copy(g_hbm.at[src_v.at[j]], rows_v[b],
                                  gsems[b]).wait()
            pltpu.sync_copy(rows_v[b], acc_sh.at[dst_v.at[j]], add=True)

        def group(gi, carry2):
            for b in range(NBUF):
                j = gi * NBUF + b
                step(j, b)
                pltpu.async_copy(g_hbm.at[src_v.at[j + NBUF]], rows_v[b],
                                 gsems[b])
            return carry2

        lax.fori_loop(0, SGRP - 1, group, 0)
        for b in range(NBUF):
            step(SSZ - NBUF + b, b)
        return carry

    lax.fori_loop(0, NSTG, stage, 0)

    plsc.subcore_barrier()
    pltpu.sync_copy(acc_sh.at[pl.ds(s * RPT, RPT)],
                    out_hbm.at[c, pl.ds(s * RPT, RPT)])


_R = 2000


def _lin_body(x_ref, w_ref, degp_ref, h_ref, g_ref, dinv_ref):
    deg = jnp.sum(degp_ref[...], axis=(0, 2)) * (1.0 / DEGW) + 1.0
    dv = lax.rsqrt(deg)[:, None]
    dinv_ref[...] = dv
    h = jnp.dot(x_ref[...], w_ref[...], preferred_element_type=jnp.float32)
    h_ref[...] = h
    g_ref[...] = h * dv


def _tc_layer_in(x, w, deg_p):
    return pl.pallas_call(
        _lin_body,
        grid=(N // _R,),
        in_specs=[
            pl.BlockSpec((_R, D), lambda i: (i, 0)),
            pl.BlockSpec((D, D), lambda i: (0, 0)),
            pl.BlockSpec((NC, _R, DEGW), lambda i: (0, i, 0)),
        ],
        out_specs=[
            pl.BlockSpec((_R, D), lambda i: (i, 0)),
            pl.BlockSpec((_R, D), lambda i: (i, 0)),
            pl.BlockSpec((_R, 1), lambda i: (i, 0)),
        ],
        out_shape=[
            jax.ShapeDtypeStruct((N, D), jnp.float32),
            jax.ShapeDtypeStruct((N, D), jnp.float32),
            jax.ShapeDtypeStruct((N, 1), jnp.float32),
        ],
    )(x, w, deg_p)


def _mid_body(p_ref, h1_ref, dinv_ref, w_ref, h2_ref, g2_ref):
    dv = dinv_ref[...]
    agg = p_ref[0] + p_ref[1]
    z = jnp.maximum(dv * agg + dv * dv * h1_ref[...], 0.0)
    h2 = jnp.dot(z, w_ref[...], preferred_element_type=jnp.float32)
    h2_ref[...] = h2
    g2_ref[...] = dv * h2


def _tc_mid(p, h1, dinv_col, w2):
    return pl.pallas_call(
        _mid_body,
        grid=(N // _R,),
        in_specs=[
            pl.BlockSpec((NC, _R, D), lambda i: (0, i, 0)),
            pl.BlockSpec((_R, D), lambda i: (i, 0)),
            pl.BlockSpec((_R, 1), lambda i: (i, 0)),
            pl.BlockSpec((D, D), lambda i: (0, 0)),
        ],
        out_specs=[
            pl.BlockSpec((_R, D), lambda i: (i, 0)),
            pl.BlockSpec((_R, D), lambda i: (i, 0)),
        ],
        out_shape=[
            jax.ShapeDtypeStruct((N, D), jnp.float32),
            jax.ShapeDtypeStruct((N, D), jnp.float32),
        ],
    )(p, h1, dinv_col, w2)


def _out_body(p_ref, h2_ref, dinv_ref, b_ref, o_ref):
    dv = dinv_ref[...]
    o_ref[...] = dv * (p_ref[0] + p_ref[1]) + dv * dv * h2_ref[...] + b_ref[...]


def _tc_out(p, h2, dinv_col, b):
    return pl.pallas_call(
        _out_body,
        grid=(N // _R,),
        in_specs=[
            pl.BlockSpec((NC, _R, D), lambda i: (0, i, 0)),
            pl.BlockSpec((_R, D), lambda i: (i, 0)),
            pl.BlockSpec((_R, 1), lambda i: (i, 0)),
            pl.BlockSpec((1, D), lambda i: (0, 0)),
        ],
        out_specs=pl.BlockSpec((_R, D), lambda i: (i, 0)),
        out_shape=jax.ShapeDtypeStruct((N, D), jnp.float32),
    )(p, h2, dinv_col, b)


def kernel(x, edge_index, W1, W2, b2):
    E = edge_index.shape[1]
    pad = E_PAD - E
    src = jnp.concatenate(
        [edge_index[0].astype(jnp.int32),
         jnp.arange(pad, dtype=jnp.int32) % N])
    dst = jnp.concatenate(
        [edge_index[1].astype(jnp.int32),
         N + (jnp.arange(pad, dtype=jnp.int32) % (ACC_N - N))])
    src2d = src.reshape(NW * NCH, CHUNK)
    dst2d = dst.reshape(NW * NCH, CHUNK)
    zeros = jnp.zeros((320, D), jnp.float32)
    ones_deg = jnp.ones((CHUNK, DEGW), jnp.float32)

    deg_p = _sc_degree(dst2d, ones_deg, zeros)
    h1, g1, dinv_col = _tc_layer_in(x, W1, deg_p)
    p1 = _sc_aggregate(g1, src2d, dst2d, zeros)
    h2, g2 = _tc_mid(p1, h1, dinv_col, W2)
    p2 = _sc_aggregate(g2, src2d, dst2d, zeros)
    out = _tc_out(p2, h2, dinv_col, b2.reshape(1, D))
    return out
